# Initial kernel scaffold; baseline (speedup 1.0000x reference)
import jax
import jax.numpy as jnp
from jax.experimental import pallas as pl


def kernel(x, t, edge_index, edge_weight, params):
    raise NotImplementedError("write your pallas kernel here")


# speedup vs baseline: 4.4106x; 4.4106x over previous
import functools

import jax
import jax.numpy as jnp
import numpy as np
from jax import lax
from jax.experimental import pallas as pl
from jax.experimental.pallas import tpu as pltpu
from jax.experimental.pallas import tpu_sc as plsc

N = 10000
C = 128
E = 320000
GROUPS = 8
GSIZE = C // GROUPS
EPS = 1e-5
SLOPE = 0.01


RB = 1000
GRID = N // RB

_POOL = np.zeros((C, GROUPS), np.float32)
for _c in range(C):
    _POOL[_c, _c // GSIZE] = 1.0 / GSIZE
_BCAST = (_POOL.T > 0).astype(np.float32)


def _gn_act(s, gamma, beta, pool, bcast):
    mean = jnp.dot(s, pool, preferred_element_type=jnp.float32)
    m2 = jnp.dot(s * s, pool, preferred_element_type=jnp.float32)
    mean_b = jnp.dot(mean, bcast, preferred_element_type=jnp.float32)
    m2_b = jnp.dot(m2, bcast, preferred_element_type=jnp.float32)
    var = m2_b - mean_b * mean_b
    xn = (s - mean_b) * lax.rsqrt(var + EPS)
    y = xn * gamma + beta
    return jnp.where(y >= 0, y, SLOPE * y)


def _row_spec():
    return pl.BlockSpec((RB, C), lambda i: (i, 0))


def _full_spec(shape):
    nd = len(shape)
    return pl.BlockSpec(shape, lambda i: (0,) * nd)


def _agg_spec():
    return pl.BlockSpec((2, RB, C), lambda i: (0, i, 0))


def _pre_shape():
    return jax.ShapeDtypeStruct((N, C), jnp.float32)


def _stage_first_body(x_ref, g_ref, b_ref, w_ref, pool_ref, bc_ref, out_ref):
    a = _gn_act(x_ref[...], g_ref[...], b_ref[...], pool_ref[...], bc_ref[...])
    out_ref[...] = jnp.dot(a, w_ref[...], preferred_element_type=jnp.float32)


def _stage_first(x, gn_g, gn_b, w, pool, bcast):
    return pl.pallas_call(
        _stage_first_body,
        grid=(GRID,),
        in_specs=[_row_spec(), _full_spec((1, C)), _full_spec((1, C)),
                  _full_spec((C, C)), _full_spec((C, GROUPS)),
                  _full_spec((GROUPS, C))],
        out_specs=_row_spec(),
        out_shape=_pre_shape(),
    )(x, gn_g, gn_b, w, pool, bcast)


def _stage_mid_body(agg_ref, t_ref, bp_ref, wt_ref, bt_ref, g_ref, b_ref,
                    w_ref, pool_ref, bc_ref, out_ref):
    ta = t_ref[...]
    ta = jnp.where(ta >= 0, ta, SLOPE * ta)
    tt = jnp.dot(ta, wt_ref[...], preferred_element_type=jnp.float32)
    s = agg_ref[0] + agg_ref[1] + bp_ref[...] + tt + bt_ref[...]
    a = _gn_act(s, g_ref[...], b_ref[...], pool_ref[...], bc_ref[...])
    out_ref[...] = jnp.dot(a, w_ref[...], preferred_element_type=jnp.float32)


def _stage_mid(agg, t, b_prev, wt, bt, gn_g, gn_b, w, pool, bcast):
    return pl.pallas_call(
        _stage_mid_body,
        grid=(GRID,),
        in_specs=[_agg_spec(), _row_spec(), _full_spec((1, C)),
                  _full_spec((C, C)), _full_spec((1, C)), _full_spec((1, C)),
                  _full_spec((1, C)), _full_spec((C, C)),
                  _full_spec((C, GROUPS)), _full_spec((GROUPS, C))],
        out_specs=_row_spec(),
        out_shape=_pre_shape(),
    )(agg, t, b_prev, wt, bt, gn_g, gn_b, w, pool, bcast)


def _stage_fuse_body(n_extra, agg_ref, bp_ref, *rest):
    extras = rest[:n_extra]
    g_ref, b_ref, w_ref, pool_ref, bc_ref, out_ref, s_ref = rest[n_extra:]
    s = agg_ref[0] + agg_ref[1] + bp_ref[...]
    for e in extras:
        s = s + e[...]
    s_ref[...] = s
    a = _gn_act(s, g_ref[...], b_ref[...], pool_ref[...], bc_ref[...])
    out_ref[...] = jnp.dot(a, w_ref[...], preferred_element_type=jnp.float32)


def _stage_fuse(agg, b_prev, extras, gn_g, gn_b, w, pool, bcast):
    n_extra = len(extras)
    return pl.pallas_call(
        functools.partial(_stage_fuse_body, n_extra),
        grid=(GRID,),
        in_specs=([_agg_spec(), _full_spec((1, C))]
                  + [_row_spec()] * n_extra
                  + [_full_spec((1, C)), _full_spec((1, C)),
                     _full_spec((C, C)), _full_spec((C, GROUPS)),
                     _full_spec((GROUPS, C))]),
        out_specs=[_row_spec(), _row_spec()],
        out_shape=[_pre_shape(), jax.ShapeDtypeStruct((N, C), jnp.float32)],
    )(agg, b_prev, *extras, gn_g, gn_b, w, pool, bcast)


def _stage_final_body(agg_ref, bp_ref, x_ref, out_ref):
    out_ref[...] = agg_ref[0] + agg_ref[1] + bp_ref[...] + x_ref[...]


def _stage_final(agg, b_prev, x):
    return pl.pallas_call(
        _stage_final_body,
        grid=(GRID,),
        in_specs=[_agg_spec(), _full_spec((1, C)), _row_spec()],
        out_specs=_row_spec(),
        out_shape=jax.ShapeDtypeStruct((N, C), jnp.float32),
    )(agg, b_prev, x)


CB = 125
NSUB = 8
BIG = CB * NSUB
NTILE = 16
E_TILE = E // (2 * NTILE)
NBIG = E_TILE // BIG
R2 = E_TILE // CB
NPAD = 10240
R_TILE = NPAD // NTILE
WPAD = 2048


def _sc_conv_body(pre_hbm, src_hbm, dst_hbm, w_hbm, zeros_hbm, out_hbm,
                  sidx, didx, wbuf, rows, acc, sem):
    c = lax.axis_index("c")
    s = lax.axis_index("s")

    pltpu.sync_copy(zeros_hbm, acc.at[pl.ds(s * R_TILE, R_TILE), :])
    plsc.subcore_barrier()

    wid = c * NTILE + s

    def big_chunk(g):
        r0 = wid * R2 + g * NSUB
        e0 = wid * E_TILE + g * BIG
        pltpu.sync_copy(src_hbm.at[pl.ds(r0, NSUB), :], sidx)
        pltpu.sync_copy(dst_hbm.at[pl.ds(r0, NSUB), :], didx)
        for j in range(NSUB):
            pltpu.sync_copy(w_hbm.at[r0 + j], wbuf)
            pltpu.async_copy(pre_hbm.at[sidx.at[j]], rows, sem).wait()

            def edge(e):
                wv = wbuf[0, pl.ds(e * 16, 16)]
                for k in range(C // 16):
                    sl = rows[e, pl.ds(k * 16, 16)]
                    rows[e, pl.ds(k * 16, 16)] = sl * wv

            pl.loop(0, CB)(edge)
            pltpu.sync_copy(rows, acc.at[didx.at[j]], add=True)

    pl.loop(0, NBIG)(big_chunk)
    plsc.subcore_barrier()

    row0 = c * NPAD + s * R_TILE
    pltpu.sync_copy(acc.at[pl.ds(s * R_TILE, R_TILE), :],
                    out_hbm.at[pl.ds(row0, R_TILE), :])


@functools.cache
def _sc_conv_kernel():
    mesh = plsc.VectorSubcoreMesh(core_axis_name="c", subcore_axis_name="s",
                                  num_cores=2, num_subcores=NTILE)
    return pl.kernel(
        _sc_conv_body,
        out_type=jax.ShapeDtypeStruct((2 * NPAD, C), jnp.float32),
        mesh=mesh,
        scratch_types=[
            pltpu.VMEM((NSUB, CB), jnp.int32),
            pltpu.VMEM((NSUB, CB), jnp.int32),
            pltpu.VMEM((1, WPAD), jnp.float32),
            pltpu.VMEM((CB, C), jnp.float32),
            pltpu.VMEM_SHARED((NPAD, C), jnp.float32),
            pltpu.SemaphoreType.DMA,
        ],
    )


def kernel(x, t, edge_index, edge_weight, params):
    pool = jnp.asarray(_POOL)
    bcast = jnp.asarray(_BCAST)
    src2 = edge_index[0].astype(jnp.int32).reshape(E // CB, CB)
    dst2 = edge_index[1].astype(jnp.int32).reshape(E // CB, CB)
    wr = jnp.broadcast_to(edge_weight.astype(jnp.float32)[:, None],
                          (E, 16)).reshape(E // CB, CB * 16)
    w = jnp.pad(wr, ((0, 0), (0, WPAD - CB * 16)))[:, None, :]
    zeros = jnp.zeros((R_TILE, C), jnp.float32)

    def row(v):
        return v.reshape(1, C)

    def conv(pre):
        agg = _sc_conv_kernel()(pre, src2, dst2, w, zeros)
        return agg.reshape(2, NPAD, C)

    p0, p1, p2, p3 = params

    pre = _stage_first(x, row(p0['gn1_g']), row(p0['gn1_b']), p0['W1'],
                       pool, bcast)
    a = conv(pre)
    pre = _stage_mid(a, t, row(p0['b1']), p0['Wt'], row(p0['bt']),
                     row(p0['gn2_g']), row(p0['gn2_b']), p0['W2'], pool, bcast)
    a = conv(pre)
    pre, h1 = _stage_fuse(a, row(p0['b2']), [x], row(p1['gn1_g']),
                          row(p1['gn1_b']), p1['W1'], pool, bcast)
    a = conv(pre)
    pre = _stage_mid(a, t, row(p1['b1']), p1['Wt'], row(p1['bt']),
                     row(p1['gn2_g']), row(p1['gn2_b']), p1['W2'], pool, bcast)
    a = conv(pre)
    pre, h2 = _stage_fuse(a, row(p1['b2']), [h1], row(p2['gn1_g']),
                          row(p2['gn1_b']), p2['W1'], pool, bcast)
    a = conv(pre)
    pre = _stage_mid(a, t, row(p2['b1']), p2['Wt'], row(p2['bt']),
                     row(p2['gn2_g']), row(p2['gn2_b']), p2['W2'], pool, bcast)
    a = conv(pre)
    pre, in4 = _stage_fuse(a, row(p2['b2']), [h2, h1], row(p3['gn1_g']),
                           row(p3['gn1_b']), p3['W1'], pool, bcast)
    a = conv(pre)
    pre = _stage_mid(a, t, row(p3['b1']), p3['Wt'], row(p3['bt']),
                     row(p3['gn2_g']), row(p3['gn2_b']), p3['W2'], pool, bcast)
    a = conv(pre)
    return _stage_final(a, row(p3['b2']), in4)


# speedup vs baseline: 5.4578x; 1.2374x over previous
import functools

import jax
import jax.numpy as jnp
import numpy as np
from jax import lax
from jax.experimental import pallas as pl
from jax.experimental.pallas import tpu as pltpu
from jax.experimental.pallas import tpu_sc as plsc

N = 10000
C = 128
E = 320000
GROUPS = 8
GSIZE = C // GROUPS
EPS = 1e-5
SLOPE = 0.01


RB = 1000
GRID = N // RB

_POOL = np.zeros((C, GROUPS), np.float32)
for _c in range(C):
    _POOL[_c, _c // GSIZE] = 1.0 / GSIZE
_BCAST = (_POOL.T > 0).astype(np.float32)


def _gn_act(s, gamma, beta, pool, bcast):
    mean = jnp.dot(s, pool, preferred_element_type=jnp.float32)
    m2 = jnp.dot(s * s, pool, preferred_element_type=jnp.float32)
    mean_b = jnp.dot(mean, bcast, preferred_element_type=jnp.float32)
    m2_b = jnp.dot(m2, bcast, preferred_element_type=jnp.float32)
    var = m2_b - mean_b * mean_b
    xn = (s - mean_b) * lax.rsqrt(var + EPS)
    y = xn * gamma + beta
    return jnp.where(y >= 0, y, SLOPE * y)


def _row_spec():
    return pl.BlockSpec((RB, C), lambda i: (i, 0))


def _full_spec(shape):
    nd = len(shape)
    return pl.BlockSpec(shape, lambda i: (0,) * nd)


def _agg_spec():
    return pl.BlockSpec((2, RB, C), lambda i: (0, i, 0))


def _pre_shape():
    return jax.ShapeDtypeStruct((N, C), jnp.float32)


def _stage_first_body(x_ref, g_ref, b_ref, w_ref, pool_ref, bc_ref, out_ref):
    a = _gn_act(x_ref[...], g_ref[...], b_ref[...], pool_ref[...], bc_ref[...])
    out_ref[...] = jnp.dot(a, w_ref[...], preferred_element_type=jnp.float32)


def _stage_first(x, gn_g, gn_b, w, pool, bcast):
    return pl.pallas_call(
        _stage_first_body,
        grid=(GRID,),
        in_specs=[_row_spec(), _full_spec((1, C)), _full_spec((1, C)),
                  _full_spec((C, C)), _full_spec((C, GROUPS)),
                  _full_spec((GROUPS, C))],
        out_specs=_row_spec(),
        out_shape=_pre_shape(),
    )(x, gn_g, gn_b, w, pool, bcast)


def _stage_mid_body(agg_ref, t_ref, bp_ref, wt_ref, bt_ref, g_ref, b_ref,
                    w_ref, pool_ref, bc_ref, out_ref):
    ta = t_ref[...]
    ta = jnp.where(ta >= 0, ta, SLOPE * ta)
    tt = jnp.dot(ta, wt_ref[...], preferred_element_type=jnp.float32)
    s = agg_ref[0] + agg_ref[1] + bp_ref[...] + tt + bt_ref[...]
    a = _gn_act(s, g_ref[...], b_ref[...], pool_ref[...], bc_ref[...])
    out_ref[...] = jnp.dot(a, w_ref[...], preferred_element_type=jnp.float32)


def _stage_mid(agg, t, b_prev, wt, bt, gn_g, gn_b, w, pool, bcast):
    return pl.pallas_call(
        _stage_mid_body,
        grid=(GRID,),
        in_specs=[_agg_spec(), _row_spec(), _full_spec((1, C)),
                  _full_spec((C, C)), _full_spec((1, C)), _full_spec((1, C)),
                  _full_spec((1, C)), _full_spec((C, C)),
                  _full_spec((C, GROUPS)), _full_spec((GROUPS, C))],
        out_specs=_row_spec(),
        out_shape=_pre_shape(),
    )(agg, t, b_prev, wt, bt, gn_g, gn_b, w, pool, bcast)


def _stage_fuse_body(n_extra, agg_ref, bp_ref, *rest):
    extras = rest[:n_extra]
    g_ref, b_ref, w_ref, pool_ref, bc_ref, out_ref, s_ref = rest[n_extra:]
    s = agg_ref[0] + agg_ref[1] + bp_ref[...]
    for e in extras:
        s = s + e[...]
    s_ref[...] = s
    a = _gn_act(s, g_ref[...], b_ref[...], pool_ref[...], bc_ref[...])
    out_ref[...] = jnp.dot(a, w_ref[...], preferred_element_type=jnp.float32)


def _stage_fuse(agg, b_prev, extras, gn_g, gn_b, w, pool, bcast):
    n_extra = len(extras)
    return pl.pallas_call(
        functools.partial(_stage_fuse_body, n_extra),
        grid=(GRID,),
        in_specs=([_agg_spec(), _full_spec((1, C))]
                  + [_row_spec()] * n_extra
                  + [_full_spec((1, C)), _full_spec((1, C)),
                     _full_spec((C, C)), _full_spec((C, GROUPS)),
                     _full_spec((GROUPS, C))]),
        out_specs=[_row_spec(), _row_spec()],
        out_shape=[_pre_shape(), jax.ShapeDtypeStruct((N, C), jnp.float32)],
    )(agg, b_prev, *extras, gn_g, gn_b, w, pool, bcast)


def _stage_final_body(agg_ref, bp_ref, x_ref, out_ref):
    out_ref[...] = agg_ref[0] + agg_ref[1] + bp_ref[...] + x_ref[...]


def _stage_final(agg, b_prev, x):
    return pl.pallas_call(
        _stage_final_body,
        grid=(GRID,),
        in_specs=[_agg_spec(), _full_spec((1, C)), _row_spec()],
        out_specs=_row_spec(),
        out_shape=jax.ShapeDtypeStruct((N, C), jnp.float32),
    )(agg, b_prev, x)


CB = 50
NSUB = 8
BIG = CB * NSUB
NTILE = 16
E_TILE = E // (2 * NTILE)
NBIG = E_TILE // BIG
R2 = E_TILE // CB
NPAD = 10240
R_TILE = NPAD // NTILE
WPAD = 896


def _sc_conv_body(pre_hbm, src_hbm, dst_hbm, w_hbm, zeros_hbm, out_hbm,
                  sidx, didx, wbuf, rows_a, rows_b, acc, sem_g, sem_s):
    c = lax.axis_index("c")
    s = lax.axis_index("s")

    pltpu.sync_copy(zeros_hbm, acc.at[pl.ds(s * R_TILE, R_TILE), :])
    plsc.subcore_barrier()

    wid = c * NTILE + s
    rows = (rows_a, rows_b)

    def big_chunk(g):
        r0 = wid * R2 + g * NSUB
        pltpu.sync_copy(src_hbm.at[pl.ds(r0, NSUB), :], sidx)
        pltpu.sync_copy(dst_hbm.at[pl.ds(r0, NSUB), :], didx)
        pltpu.sync_copy(w_hbm.at[wid * NBIG + g], wbuf)
        gat = {}
        scat = {}
        gat[0] = pltpu.async_copy(pre_hbm.at[sidx.at[0]], rows[0], sem_g)
        for j in range(NSUB):
            b = j & 1
            gat[j].wait()
            if j + 1 < NSUB:
                if j >= 1:
                    scat[j - 1].wait()
                gat[j + 1] = pltpu.async_copy(
                    pre_hbm.at[sidx.at[j + 1]], rows[1 - b], sem_g)
            rb = rows[b]

            def edge(e):
                wv = wbuf[j, pl.ds(e * 16, 16)]
                for k in range(C // 16):
                    sl = rb[e, pl.ds(k * 16, 16)]
                    rb[e, pl.ds(k * 16, 16)] = sl * wv

            plsc.parallel_loop(0, CB, unroll=2)(edge)
            scat[j] = pltpu.async_copy(rb, acc.at[didx.at[j]], sem_s,
                                       add=True)
        scat[NSUB - 2].wait()
        scat[NSUB - 1].wait()

    pl.loop(0, NBIG)(big_chunk)
    plsc.subcore_barrier()

    row0 = c * NPAD + s * R_TILE
    pltpu.sync_copy(acc.at[pl.ds(s * R_TILE, R_TILE), :],
                    out_hbm.at[pl.ds(row0, R_TILE), :])


@functools.cache
def _sc_conv_kernel():
    mesh = plsc.VectorSubcoreMesh(core_axis_name="c", subcore_axis_name="s",
                                  num_cores=2, num_subcores=NTILE)
    return pl.kernel(
        _sc_conv_body,
        out_type=jax.ShapeDtypeStruct((2 * NPAD, C), jnp.float32),
        mesh=mesh,
        scratch_types=[
            pltpu.VMEM((NSUB, CB), jnp.int32),
            pltpu.VMEM((NSUB, CB), jnp.int32),
            pltpu.VMEM((NSUB, WPAD), jnp.float32),
            pltpu.VMEM((CB, C), jnp.float32),
            pltpu.VMEM((CB, C), jnp.float32),
            pltpu.VMEM_SHARED((NPAD, C), jnp.float32),
            pltpu.SemaphoreType.DMA,
            pltpu.SemaphoreType.DMA,
        ],
    )


def kernel(x, t, edge_index, edge_weight, params):
    pool = jnp.asarray(_POOL)
    bcast = jnp.asarray(_BCAST)
    src2 = edge_index[0].astype(jnp.int32).reshape(E // CB, CB)
    dst2 = edge_index[1].astype(jnp.int32).reshape(E // CB, CB)
    wr = jnp.broadcast_to(edge_weight.astype(jnp.float32)[:, None],
                          (E, 16)).reshape(E // CB, CB * 16)
    w = jnp.pad(wr, ((0, 0), (0, WPAD - CB * 16))).reshape(
        E // BIG, NSUB, WPAD)
    zeros = jnp.zeros((R_TILE, C), jnp.float32)

    def row(v):
        return v.reshape(1, C)

    def conv(pre):
        agg = _sc_conv_kernel()(pre, src2, dst2, w, zeros)
        return agg.reshape(2, NPAD, C)

    p0, p1, p2, p3 = params

    pre = _stage_first(x, row(p0['gn1_g']), row(p0['gn1_b']), p0['W1'],
                       pool, bcast)
    a = conv(pre)
    pre = _stage_mid(a, t, row(p0['b1']), p0['Wt'], row(p0['bt']),
                     row(p0['gn2_g']), row(p0['gn2_b']), p0['W2'], pool, bcast)
    a = conv(pre)
    pre, h1 = _stage_fuse(a, row(p0['b2']), [x], row(p1['gn1_g']),
                          row(p1['gn1_b']), p1['W1'], pool, bcast)
    a = conv(pre)
    pre = _stage_mid(a, t, row(p1['b1']), p1['Wt'], row(p1['bt']),
                     row(p1['gn2_g']), row(p1['gn2_b']), p1['W2'], pool, bcast)
    a = conv(pre)
    pre, h2 = _stage_fuse(a, row(p1['b2']), [h1], row(p2['gn1_g']),
                          row(p2['gn1_b']), p2['W1'], pool, bcast)
    a = conv(pre)
    pre = _stage_mid(a, t, row(p2['b1']), p2['Wt'], row(p2['bt']),
                     row(p2['gn2_g']), row(p2['gn2_b']), p2['W2'], pool, bcast)
    a = conv(pre)
    pre, in4 = _stage_fuse(a, row(p2['b2']), [h2, h1], row(p3['gn1_g']),
                           row(p3['gn1_b']), p3['W1'], pool, bcast)
    a = conv(pre)
    pre = _stage_mid(a, t, row(p3['b1']), p3['Wt'], row(p3['bt']),
                     row(p3['gn2_g']), row(p3['gn2_b']), p3['W2'], pool, bcast)
    a = conv(pre)
    return _stage_final(a, row(p3['b2']), in4)


# speedup vs baseline: 7.0341x; 1.2888x over previous
import functools

import jax
import jax.numpy as jnp
import numpy as np
from jax import lax
from jax.experimental import pallas as pl
from jax.experimental.pallas import tpu as pltpu
from jax.experimental.pallas import tpu_sc as plsc

N = 10000
C = 128
E = 320000
GROUPS = 8
GSIZE = C // GROUPS
EPS = 1e-5
SLOPE = 0.01


RB = 1000
GRID = N // RB

_POOL = np.zeros((C, GROUPS), np.float32)
for _c in range(C):
    _POOL[_c, _c // GSIZE] = 1.0 / GSIZE
_BCAST = (_POOL.T > 0).astype(np.float32)


def _gn_act(s, gamma, beta, pool, bcast):
    mean = jnp.dot(s, pool, preferred_element_type=jnp.float32)
    m2 = jnp.dot(s * s, pool, preferred_element_type=jnp.float32)
    mean_b = jnp.dot(mean, bcast, preferred_element_type=jnp.float32)
    m2_b = jnp.dot(m2, bcast, preferred_element_type=jnp.float32)
    var = m2_b - mean_b * mean_b
    xn = (s - mean_b) * lax.rsqrt(var + EPS)
    y = xn * gamma + beta
    return jnp.where(y >= 0, y, SLOPE * y)


def _row_spec():
    return pl.BlockSpec((RB, C), lambda i: (i, 0))


def _full_spec(shape):
    nd = len(shape)
    return pl.BlockSpec(shape, lambda i: (0,) * nd)


def _agg_spec():
    return pl.BlockSpec((2, RB, C), lambda i: (0, i, 0))


def _pre_shape():
    return jax.ShapeDtypeStruct((N, C), jnp.float32)


def _stage_first_body(x_ref, g_ref, b_ref, w_ref, pool_ref, bc_ref, out_ref):
    a = _gn_act(x_ref[...], g_ref[...], b_ref[...], pool_ref[...], bc_ref[...])
    out_ref[...] = jnp.dot(a, w_ref[...], preferred_element_type=jnp.float32)


def _stage_first(x, gn_g, gn_b, w, pool, bcast):
    return pl.pallas_call(
        _stage_first_body,
        grid=(GRID,),
        in_specs=[_row_spec(), _full_spec((1, C)), _full_spec((1, C)),
                  _full_spec((C, C)), _full_spec((C, GROUPS)),
                  _full_spec((GROUPS, C))],
        out_specs=_row_spec(),
        out_shape=_pre_shape(),
    )(x, gn_g, gn_b, w, pool, bcast)


def _stage_mid_body(agg_ref, t_ref, bp_ref, wt_ref, bt_ref, g_ref, b_ref,
                    w_ref, pool_ref, bc_ref, out_ref):
    ta = t_ref[...]
    ta = jnp.where(ta >= 0, ta, SLOPE * ta)
    tt = jnp.dot(ta, wt_ref[...], preferred_element_type=jnp.float32)
    s = agg_ref[0] + agg_ref[1] + bp_ref[...] + tt + bt_ref[...]
    a = _gn_act(s, g_ref[...], b_ref[...], pool_ref[...], bc_ref[...])
    out_ref[...] = jnp.dot(a, w_ref[...], preferred_element_type=jnp.float32)


def _stage_mid(agg, t, b_prev, wt, bt, gn_g, gn_b, w, pool, bcast):
    return pl.pallas_call(
        _stage_mid_body,
        grid=(GRID,),
        in_specs=[_agg_spec(), _row_spec(), _full_spec((1, C)),
                  _full_spec((C, C)), _full_spec((1, C)), _full_spec((1, C)),
                  _full_spec((1, C)), _full_spec((C, C)),
                  _full_spec((C, GROUPS)), _full_spec((GROUPS, C))],
        out_specs=_row_spec(),
        out_shape=_pre_shape(),
    )(agg, t, b_prev, wt, bt, gn_g, gn_b, w, pool, bcast)


def _stage_fuse_body(n_extra, agg_ref, bp_ref, *rest):
    extras = rest[:n_extra]
    g_ref, b_ref, w_ref, pool_ref, bc_ref, out_ref, s_ref = rest[n_extra:]
    s = agg_ref[0] + agg_ref[1] + bp_ref[...]
    for e in extras:
        s = s + e[...]
    s_ref[...] = s
    a = _gn_act(s, g_ref[...], b_ref[...], pool_ref[...], bc_ref[...])
    out_ref[...] = jnp.dot(a, w_ref[...], preferred_element_type=jnp.float32)


def _stage_fuse(agg, b_prev, extras, gn_g, gn_b, w, pool, bcast):
    n_extra = len(extras)
    return pl.pallas_call(
        functools.partial(_stage_fuse_body, n_extra),
        grid=(GRID,),
        in_specs=([_agg_spec(), _full_spec((1, C))]
                  + [_row_spec()] * n_extra
                  + [_full_spec((1, C)), _full_spec((1, C)),
                     _full_spec((C, C)), _full_spec((C, GROUPS)),
                     _full_spec((GROUPS, C))]),
        out_specs=[_row_spec(), _row_spec()],
        out_shape=[_pre_shape(), jax.ShapeDtypeStruct((N, C), jnp.float32)],
    )(agg, b_prev, *extras, gn_g, gn_b, w, pool, bcast)


def _stage_final_body(agg_ref, bp_ref, x_ref, out_ref):
    out_ref[...] = agg_ref[0] + agg_ref[1] + bp_ref[...] + x_ref[...]


def _stage_final(agg, b_prev, x):
    return pl.pallas_call(
        _stage_final_body,
        grid=(GRID,),
        in_specs=[_agg_spec(), _full_spec((1, C)), _row_spec()],
        out_specs=_row_spec(),
        out_shape=jax.ShapeDtypeStruct((N, C), jnp.float32),
    )(agg, b_prev, x)


CB = 50
NSUB = 8
BIG = CB * NSUB
NTILE = 16
E_TILE = E // (2 * NTILE)
NBIG = E_TILE // BIG
R2 = E_TILE // CB
NPAD = 10240
R_TILE = NPAD // NTILE
WPAD = 896


def _sc_conv_body(pre_hbm, src_hbm, dst_hbm, w_hbm, zeros_hbm, out_hbm,
                  sidx, didx, wbuf, rows_a, rows_b, rows_c, acc,
                  sem_g, sem_s):
    c = lax.axis_index("c")
    s = lax.axis_index("s")

    pltpu.sync_copy(zeros_hbm, acc.at[pl.ds(s * R_TILE, R_TILE), :])
    plsc.subcore_barrier()

    wid = c * NTILE + s
    rows = (rows_a, rows_b, rows_c)

    def big_chunk(g):
        r0 = wid * R2 + g * NSUB
        pltpu.sync_copy(src_hbm.at[pl.ds(r0, NSUB), :], sidx)
        pltpu.sync_copy(dst_hbm.at[pl.ds(r0, NSUB), :], didx)
        pltpu.sync_copy(w_hbm.at[wid * NBIG + g], wbuf)
        gat = {}
        scat = {}
        gat[0] = pltpu.async_copy(pre_hbm.at[sidx.at[0]], rows[0], sem_g)
        gat[1] = pltpu.async_copy(pre_hbm.at[sidx.at[1]], rows[1], sem_g)
        for j in range(NSUB):
            rb = rows[j % 3]
            gat[j].wait()
            if j + 2 < NSUB:
                if j >= 1:
                    scat[j - 1].wait()
                gat[j + 2] = pltpu.async_copy(
                    pre_hbm.at[sidx.at[j + 2]], rows[(j + 2) % 3], sem_g)

            def edge(e):
                wv = wbuf[j, pl.ds(e * 16, 16)]
                for k in range(C // 16):
                    sl = rb[e, pl.ds(k * 16, 16)]
                    rb[e, pl.ds(k * 16, 16)] = sl * wv

            plsc.parallel_loop(0, CB, unroll=2)(edge)
            scat[j] = pltpu.async_copy(rb, acc.at[didx.at[j]], sem_s,
                                       add=True)
        scat[NSUB - 3].wait()
        scat[NSUB - 2].wait()
        scat[NSUB - 1].wait()

    pl.loop(0, NBIG)(big_chunk)
    plsc.subcore_barrier()

    row0 = c * NPAD + s * R_TILE
    pltpu.sync_copy(acc.at[pl.ds(s * R_TILE, R_TILE), :],
                    out_hbm.at[pl.ds(row0, R_TILE), :])


@functools.cache
def _sc_conv_kernel():
    mesh = plsc.VectorSubcoreMesh(core_axis_name="c", subcore_axis_name="s",
                                  num_cores=2, num_subcores=NTILE)
    return pl.kernel(
        _sc_conv_body,
        out_type=jax.ShapeDtypeStruct((2 * NPAD, C), jnp.float32),
        mesh=mesh,
        scratch_types=[
            pltpu.VMEM((NSUB, CB), jnp.int32),
            pltpu.VMEM((NSUB, CB), jnp.int32),
            pltpu.VMEM((NSUB, WPAD), jnp.float32),
            pltpu.VMEM((CB, C), jnp.float32),
            pltpu.VMEM((CB, C), jnp.float32),
            pltpu.VMEM((CB, C), jnp.float32),
            pltpu.VMEM_SHARED((NPAD, C), jnp.float32),
            pltpu.SemaphoreType.DMA,
            pltpu.SemaphoreType.DMA,
        ],
    )


def kernel(x, t, edge_index, edge_weight, params):
    pool = jnp.asarray(_POOL)
    bcast = jnp.asarray(_BCAST)
    src2 = edge_index[0].astype(jnp.int32).reshape(E // CB, CB)
    dst2 = edge_index[1].astype(jnp.int32).reshape(E // CB, CB)
    wr = jnp.broadcast_to(edge_weight.astype(jnp.float32)[:, None],
                          (E, 16)).reshape(E // CB, CB * 16)
    w = jnp.pad(wr, ((0, 0), (0, WPAD - CB * 16))).reshape(
        E // BIG, NSUB, WPAD)
    zeros = jnp.zeros((R_TILE, C), jnp.float32)

    def row(v):
        return v.reshape(1, C)

    def conv(pre):
        agg = _sc_conv_kernel()(pre, src2, dst2, w, zeros)
        return agg.reshape(2, NPAD, C)

    p0, p1, p2, p3 = params

    pre = _stage_first(x, row(p0['gn1_g']), row(p0['gn1_b']), p0['W1'],
                       pool, bcast)
    a = conv(pre)
    pre = _stage_mid(a, t, row(p0['b1']), p0['Wt'], row(p0['bt']),
                     row(p0['gn2_g']), row(p0['gn2_b']), p0['W2'], pool, bcast)
    a = conv(pre)
    pre, h1 = _stage_fuse(a, row(p0['b2']), [x], row(p1['gn1_g']),
                          row(p1['gn1_b']), p1['W1'], pool, bcast)
    a = conv(pre)
    pre = _stage_mid(a, t, row(p1['b1']), p1['Wt'], row(p1['bt']),
                     row(p1['gn2_g']), row(p1['gn2_b']), p1['W2'], pool, bcast)
    a = conv(pre)
    pre, h2 = _stage_fuse(a, row(p1['b2']), [h1], row(p2['gn1_g']),
                          row(p2['gn1_b']), p2['W1'], pool, bcast)
    a = conv(pre)
    pre = _stage_mid(a, t, row(p2['b1']), p2['Wt'], row(p2['bt']),
                     row(p2['gn2_g']), row(p2['gn2_b']), p2['W2'], pool, bcast)
    a = conv(pre)
    pre, in4 = _stage_fuse(a, row(p2['b2']), [h2, h1], row(p3['gn1_g']),
                           row(p3['gn1_b']), p3['W1'], pool, bcast)
    a = conv(pre)
    pre = _stage_mid(a, t, row(p3['b1']), p3['Wt'], row(p3['bt']),
                     row(p3['gn2_g']), row(p3['gn2_b']), p3['W2'], pool, bcast)
    a = conv(pre)
    return _stage_final(a, row(p3['b2']), in4)


# speedup vs baseline: 7.2845x; 1.0356x over previous
import functools

import jax
import jax.numpy as jnp
import numpy as np
from jax import lax
from jax.experimental import pallas as pl
from jax.experimental.pallas import tpu as pltpu
from jax.experimental.pallas import tpu_sc as plsc

N = 10000
C = 128
E = 320000
GROUPS = 8
GSIZE = C // GROUPS
EPS = 1e-5
SLOPE = 0.01


RB = 1000
GRID = N // RB

_POOL = np.zeros((C, GROUPS), np.float32)
for _c in range(C):
    _POOL[_c, _c // GSIZE] = 1.0 / GSIZE
_BCAST = (_POOL.T > 0).astype(np.float32)


def _gn_act(s, gamma, beta, pool, bcast):
    mean = jnp.dot(s, pool, preferred_element_type=jnp.float32)
    m2 = jnp.dot(s * s, pool, preferred_element_type=jnp.float32)
    mean_b = jnp.dot(mean, bcast, preferred_element_type=jnp.float32)
    m2_b = jnp.dot(m2, bcast, preferred_element_type=jnp.float32)
    var = m2_b - mean_b * mean_b
    xn = (s - mean_b) * lax.rsqrt(var + EPS)
    y = xn * gamma + beta
    return jnp.where(y >= 0, y, SLOPE * y)


def _row_spec():
    return pl.BlockSpec((RB, C), lambda i: (i, 0))


def _full_spec(shape):
    nd = len(shape)
    return pl.BlockSpec(shape, lambda i: (0,) * nd)


def _agg_spec():
    return pl.BlockSpec((2, RB, C), lambda i: (0, i, 0))


def _pre_shape():
    return jax.ShapeDtypeStruct((N, C), jnp.float32)


def _stage_first_body(x_ref, g_ref, b_ref, w_ref, pool_ref, bc_ref, out_ref):
    a = _gn_act(x_ref[...], g_ref[...], b_ref[...], pool_ref[...], bc_ref[...])
    out_ref[...] = jnp.dot(a, w_ref[...], preferred_element_type=jnp.float32)


def _stage_first(x, gn_g, gn_b, w, pool, bcast):
    return pl.pallas_call(
        _stage_first_body,
        grid=(GRID,),
        in_specs=[_row_spec(), _full_spec((1, C)), _full_spec((1, C)),
                  _full_spec((C, C)), _full_spec((C, GROUPS)),
                  _full_spec((GROUPS, C))],
        out_specs=_row_spec(),
        out_shape=_pre_shape(),
    )(x, gn_g, gn_b, w, pool, bcast)


def _stage_mid_body(agg_ref, t_ref, bp_ref, wt_ref, bt_ref, g_ref, b_ref,
                    w_ref, pool_ref, bc_ref, out_ref):
    ta = t_ref[...]
    ta = jnp.where(ta >= 0, ta, SLOPE * ta)
    tt = jnp.dot(ta, wt_ref[...], preferred_element_type=jnp.float32)
    s = agg_ref[0] + agg_ref[1] + bp_ref[...] + tt + bt_ref[...]
    a = _gn_act(s, g_ref[...], b_ref[...], pool_ref[...], bc_ref[...])
    out_ref[...] = jnp.dot(a, w_ref[...], preferred_element_type=jnp.float32)


def _stage_mid(agg, t, b_prev, wt, bt, gn_g, gn_b, w, pool, bcast):
    return pl.pallas_call(
        _stage_mid_body,
        grid=(GRID,),
        in_specs=[_agg_spec(), _row_spec(), _full_spec((1, C)),
                  _full_spec((C, C)), _full_spec((1, C)), _full_spec((1, C)),
                  _full_spec((1, C)), _full_spec((C, C)),
                  _full_spec((C, GROUPS)), _full_spec((GROUPS, C))],
        out_specs=_row_spec(),
        out_shape=_pre_shape(),
    )(agg, t, b_prev, wt, bt, gn_g, gn_b, w, pool, bcast)


def _stage_fuse_body(n_extra, agg_ref, bp_ref, *rest):
    extras = rest[:n_extra]
    g_ref, b_ref, w_ref, pool_ref, bc_ref, out_ref, s_ref = rest[n_extra:]
    s = agg_ref[0] + agg_ref[1] + bp_ref[...]
    for e in extras:
        s = s + e[...]
    s_ref[...] = s
    a = _gn_act(s, g_ref[...], b_ref[...], pool_ref[...], bc_ref[...])
    out_ref[...] = jnp.dot(a, w_ref[...], preferred_element_type=jnp.float32)


def _stage_fuse(agg, b_prev, extras, gn_g, gn_b, w, pool, bcast):
    n_extra = len(extras)
    return pl.pallas_call(
        functools.partial(_stage_fuse_body, n_extra),
        grid=(GRID,),
        in_specs=([_agg_spec(), _full_spec((1, C))]
                  + [_row_spec()] * n_extra
                  + [_full_spec((1, C)), _full_spec((1, C)),
                     _full_spec((C, C)), _full_spec((C, GROUPS)),
                     _full_spec((GROUPS, C))]),
        out_specs=[_row_spec(), _row_spec()],
        out_shape=[_pre_shape(), jax.ShapeDtypeStruct((N, C), jnp.float32)],
    )(agg, b_prev, *extras, gn_g, gn_b, w, pool, bcast)


def _stage_final_body(agg_ref, bp_ref, x_ref, out_ref):
    out_ref[...] = agg_ref[0] + agg_ref[1] + bp_ref[...] + x_ref[...]


def _stage_final(agg, b_prev, x):
    return pl.pallas_call(
        _stage_final_body,
        grid=(GRID,),
        in_specs=[_agg_spec(), _full_spec((1, C)), _row_spec()],
        out_specs=_row_spec(),
        out_shape=jax.ShapeDtypeStruct((N, C), jnp.float32),
    )(agg, b_prev, x)


CB = 50
NSUB = 8
BIG = CB * NSUB
NTILE = 16
E_TILE = E // (2 * NTILE)
NBIG = E_TILE // BIG
R2 = E_TILE // CB
NPAD = 10240
R_TILE = NPAD // NTILE
WPAD = 896


def _sc_conv_body(pre_hbm, src_hbm, dst_hbm, w_hbm, zeros_hbm, out_hbm,
                  sidx, didx, wbuf, rows_a, rows_b, rows_c, rows_d, acc,
                  sem_g, sem_s):
    c = lax.axis_index("c")
    s = lax.axis_index("s")

    pltpu.sync_copy(zeros_hbm, acc.at[pl.ds(s * R_TILE, R_TILE), :])
    plsc.subcore_barrier()

    wid = c * NTILE + s
    rows = (rows_a, rows_b, rows_c, rows_d)

    def big_chunk(g):
        r0 = wid * R2 + g * NSUB
        pltpu.sync_copy(src_hbm.at[pl.ds(r0, NSUB), :], sidx)
        pltpu.sync_copy(dst_hbm.at[pl.ds(r0, NSUB), :], didx)
        pltpu.sync_copy(w_hbm.at[wid * NBIG + g], wbuf)
        gat = {}
        scat = {}
        gat[0] = pltpu.async_copy(pre_hbm.at[sidx.at[0]], rows[0], sem_g)
        gat[1] = pltpu.async_copy(pre_hbm.at[sidx.at[1]], rows[1], sem_g)
        gat[2] = pltpu.async_copy(pre_hbm.at[sidx.at[2]], rows[2], sem_g)
        for j in range(NSUB):
            rb = rows[j % 4]
            gat[j].wait()
            if j + 3 < NSUB:
                if j >= 1:
                    scat[j - 1].wait()
                gat[j + 3] = pltpu.async_copy(
                    pre_hbm.at[sidx.at[j + 3]], rows[(j + 3) % 4], sem_g)

            def edge(e):
                wv = wbuf[j, pl.ds(e * 16, 16)]
                for k in range(C // 16):
                    sl = rb[e, pl.ds(k * 16, 16)]
                    rb[e, pl.ds(k * 16, 16)] = sl * wv

            plsc.parallel_loop(0, CB, unroll=4)(edge)
            scat[j] = pltpu.async_copy(rb, acc.at[didx.at[j]], sem_s,
                                       add=True)
        for j in range(NSUB - 4, NSUB):
            scat[j].wait()

    pl.loop(0, NBIG)(big_chunk)
    plsc.subcore_barrier()

    row0 = c * NPAD + s * R_TILE
    pltpu.sync_copy(acc.at[pl.ds(s * R_TILE, R_TILE), :],
                    out_hbm.at[pl.ds(row0, R_TILE), :])


@functools.cache
def _sc_conv_kernel():
    mesh = plsc.VectorSubcoreMesh(core_axis_name="c", subcore_axis_name="s",
                                  num_cores=2, num_subcores=NTILE)
    return pl.kernel(
        _sc_conv_body,
        out_type=jax.ShapeDtypeStruct((2 * NPAD, C), jnp.float32),
        mesh=mesh,
        scratch_types=[
            pltpu.VMEM((NSUB, CB), jnp.int32),
            pltpu.VMEM((NSUB, CB), jnp.int32),
            pltpu.VMEM((NSUB, WPAD), jnp.float32),
            pltpu.VMEM((CB, C), jnp.float32),
            pltpu.VMEM((CB, C), jnp.float32),
            pltpu.VMEM((CB, C), jnp.float32),
            pltpu.VMEM((CB, C), jnp.float32),
            pltpu.VMEM_SHARED((NPAD, C), jnp.float32),
            pltpu.SemaphoreType.DMA,
            pltpu.SemaphoreType.DMA,
        ],
    )


def kernel(x, t, edge_index, edge_weight, params):
    pool = jnp.asarray(_POOL)
    bcast = jnp.asarray(_BCAST)
    src2 = edge_index[0].astype(jnp.int32).reshape(E // CB, CB)
    dst2 = edge_index[1].astype(jnp.int32).reshape(E // CB, CB)
    wr = jnp.broadcast_to(edge_weight.astype(jnp.float32)[:, None],
                          (E, 16)).reshape(E // CB, CB * 16)
    w = jnp.pad(wr, ((0, 0), (0, WPAD - CB * 16))).reshape(
        E // BIG, NSUB, WPAD)
    zeros = jnp.zeros((R_TILE, C), jnp.float32)

    def row(v):
        return v.reshape(1, C)

    def conv(pre):
        agg = _sc_conv_kernel()(pre, src2, dst2, w, zeros)
        return agg.reshape(2, NPAD, C)

    p0, p1, p2, p3 = params

    pre = _stage_first(x, row(p0['gn1_g']), row(p0['gn1_b']), p0['W1'],
                       pool, bcast)
    a = conv(pre)
    pre = _stage_mid(a, t, row(p0['b1']), p0['Wt'], row(p0['bt']),
                     row(p0['gn2_g']), row(p0['gn2_b']), p0['W2'], pool, bcast)
    a = conv(pre)
    pre, h1 = _stage_fuse(a, row(p0['b2']), [x], row(p1['gn1_g']),
                          row(p1['gn1_b']), p1['W1'], pool, bcast)
    a = conv(pre)
    pre = _stage_mid(a, t, row(p1['b1']), p1['Wt'], row(p1['bt']),
                     row(p1['gn2_g']), row(p1['gn2_b']), p1['W2'], pool, bcast)
    a = conv(pre)
    pre, h2 = _stage_fuse(a, row(p1['b2']), [h1], row(p2['gn1_g']),
                          row(p2['gn1_b']), p2['W1'], pool, bcast)
    a = conv(pre)
    pre = _stage_mid(a, t, row(p2['b1']), p2['Wt'], row(p2['bt']),
                     row(p2['gn2_g']), row(p2['gn2_b']), p2['W2'], pool, bcast)
    a = conv(pre)
    pre, in4 = _stage_fuse(a, row(p2['b2']), [h2, h1], row(p3['gn1_g']),
                           row(p3['gn1_b']), p3['W1'], pool, bcast)
    a = conv(pre)
    pre = _stage_mid(a, t, row(p3['b1']), p3['Wt'], row(p3['bt']),
                     row(p3['gn2_g']), row(p3['gn2_b']), p3['W2'], pool, bcast)
    a = conv(pre)
    return _stage_final(a, row(p3['b2']), in4)
